# baseline (speedup 1.0000x reference)
import functools

import jax
import jax.numpy as jnp
from jax import lax
from jax.experimental import pallas as pl
from jax.experimental.pallas import tpu as pltpu
from jax.experimental.pallas import tpu_sc as plsc

N = 10000
E = 320000
D_IN = 128
H = 64
ENS = 10

NB = 1000
GRID_PRE = N // NB

C = 128
NW = 32
TOTCH = 2560
E_PAD = TOTCH * C
CPW0 = 80
CPW1 = 80
CPWMAX = max(CPW0, CPW1)
NBUF = 4
LOOK = 2
SUB_ROWS = 648
N_PAD = SUB_ROWS * 16


def _pre_body(ens_ref, x_ref, w1, b1, w2, b2, w3, b3, w4, b4, wdx, wde, bd,
              be0, out_ref, outb_ref):
    sacc = jnp.zeros((NB, H), jnp.float32)
    for e in range(ENS):
        m = ens_ref[e]
        sacc = sacc + jnp.maximum(
            jnp.dot(m, w1[...], preferred_element_type=jnp.float32) + b1[...],
            0.0)
    agg = (jnp.dot(sacc.astype(jnp.bfloat16), w2[...],
                   preferred_element_type=jnp.float32)
           + float(ENS) * b2[...])
    emb = jnp.maximum(jnp.dot(agg, w3[...], preferred_element_type=jnp.float32)
                      + b3[...], 0.0)
    emb = jnp.dot(emb, w4[...], preferred_element_type=jnp.float32) + b4[...]
    nf = (jnp.dot(x_ref[...], wdx[...], preferred_element_type=jnp.float32)
          + jnp.dot(emb, wde[...], preferred_element_type=jnp.float32)
          + bd[...])
    out_ref[...] = nf
    outb_ref[...] = nf + be0[...]


def _pre(ensemble, x, ds, Wd, bd, be0):
    full = lambda shape: pl.BlockSpec(shape, lambda i: (0,) * len(shape))
    return pl.pallas_call(
        _pre_body,
        grid=(GRID_PRE,),
        in_specs=[
            pl.BlockSpec((ENS, NB, D_IN), lambda i: (0, i, 0)),
            pl.BlockSpec((NB, D_IN), lambda i: (i, 0)),
            full((D_IN, H)), full((1, H)),
            full((H, H)), full((1, H)),
            full((H, H)), full((1, H)),
            full((H, H)), full((1, H)),
            full((D_IN, H)), full((H, H)), full((1, H)), full((1, H)),
        ],
        out_specs=[pl.BlockSpec((NB, H), lambda i: (i, 0)),
                   pl.BlockSpec((NB, H), lambda i: (i, 0))],
        out_shape=[jax.ShapeDtypeStruct((N, H), jnp.float32),
                   jax.ShapeDtypeStruct((N, H), jnp.float32)],
    )(ensemble.transpose(1, 0, 2).astype(jnp.bfloat16), x,
      ds['W1'].astype(jnp.bfloat16), ds['b1'].reshape(1, H),
      ds['W2'].astype(jnp.bfloat16), ds['b2'].reshape(1, H),
      ds['W3'], ds['b3'].reshape(1, H),
      ds['W4'], ds['b4'].reshape(1, H),
      Wd[:D_IN], Wd[D_IN:], bd.reshape(1, H), be0.reshape(1, H))


def _mp_body(h_hbm, src_hbm, dst_hbm, ea_hbm, wb_hbm, z_hbm, out_hbm,
             srcv, dstv, eav, rowsv, wbv, acc, gsem, ssem):
    c = lax.axis_index("c")
    s = lax.axis_index("s")

    with jax.named_scope("zero_acc"):
        pltpu.sync_copy(z_hbm.at[pl.ds(s * SUB_ROWS, SUB_ROWS)],
                        acc.at[pl.ds(s * SUB_ROWS, SUB_ROWS)])
        pltpu.sync_copy(wb_hbm, wbv)

    def fire_gather(j, b):
        pltpu.async_copy(h_hbm.at[srcv.at[j]], rowsv.at[b], gsem.at[b])

    def wait_gather(j, b):
        pltpu.make_async_copy(h_hbm.at[srcv.at[j]], rowsv.at[b],
                              gsem.at[b]).wait()

    def fire_scatter(j, b):
        pltpu.async_copy(rowsv.at[b], acc.at[dstv.at[j]], ssem.at[b],
                         add=True)

    def wait_scatter(j, b):
        pltpu.make_async_copy(rowsv.at[b], acc.at[dstv.at[j]],
                              ssem.at[b]).wait()

    we = [wbv[0, pl.ds(g * 16, 16)] for g in range(4)]

    def run(start, cpw):
        with jax.named_scope("stage_idx"):
            pltpu.sync_copy(src_hbm.at[pl.ds(start, cpw)],
                            srcv.at[pl.ds(0, cpw)])
            pltpu.sync_copy(dst_hbm.at[pl.ds(start, cpw)],
                            dstv.at[pl.ds(0, cpw)])
            pltpu.sync_copy(ea_hbm.at[pl.ds(start, cpw)],
                            eav.at[pl.ds(0, cpw)])

        for b in range(LOOK):
            fire_gather(b, b)

        with jax.named_scope("pre_barrier"):
            plsc.subcore_barrier()

        @pl.loop(0, cpw // NBUF)
        def _outer(jo):
            for b in range(NBUF):
                j = jo * NBUF + b
                bn = (b + LOOK) % NBUF

                @pl.when(j + LOOK < cpw)
                def _fire():
                    @pl.when(j + LOOK >= NBUF)
                    def _drain():
                        wait_scatter(j + LOOK - NBUF, bn)
                    fire_gather(j + LOOK, bn)

                wait_gather(j, b)

                @pl.loop(0, C // 16)
                def _blk(jj):
                    a_vec = eav[j, pl.ds(jj * 16, 16)]
                    for ii in range(16):
                        i = jj * 16 + ii
                        a = a_vec[ii]
                        for g in range(4):
                            sl = pl.ds(g * 16, 16)
                            rowsv[b, i, sl] = jnp.maximum(
                                rowsv[b, i, sl] + we[g] * a, 0.0)

                fire_scatter(j, b)

        with jax.named_scope("drain"):
            for b in range(NBUF):
                wait_scatter(cpw - NBUF + b, b)

    @pl.when(c == 0)
    def _core0():
        run(s * CPW0, CPW0)

    @pl.when(c == 1)
    def _core1():
        run(16 * CPW0 + s * CPW1, CPW1)

    with jax.named_scope("post_barrier"):
        plsc.subcore_barrier()
    with jax.named_scope("writeout"):
        pltpu.sync_copy(acc.at[pl.ds(s * SUB_ROWS, SUB_ROWS)],
                        out_hbm.at[c, pl.ds(s * SUB_ROWS, SUB_ROWS)])


@functools.lru_cache(maxsize=None)
def _mp_call():
    return pl.kernel(
        _mp_body,
        mesh=plsc.VectorSubcoreMesh(core_axis_name="c", subcore_axis_name="s"),
        out_type=jax.ShapeDtypeStruct((2, N_PAD, H), jnp.float32),
        scratch_types=[
            pltpu.VMEM((CPWMAX, C), jnp.int32),
            pltpu.VMEM((CPWMAX, C), jnp.int32),
            pltpu.VMEM((CPWMAX, C), jnp.float32),
            pltpu.VMEM((NBUF, C, H), jnp.float32),
            pltpu.VMEM((1, H), jnp.float32),
            pltpu.VMEM_SHARED((N_PAD, H), jnp.float32),
            pltpu.SemaphoreType.DMA((NBUF,)),
            pltpu.SemaphoreType.DMA((NBUF,)),
        ],
        compiler_params=pltpu.CompilerParams(use_tc_tiling_on_sc=False),
    )


def _mlp_body(first, last, h_ref, agg_ref, eps_ref, wm1, bm1, gm, bt,
              wm2, bm2, wa, ba, benext, *out_refs):
    h = h_ref[...]
    z = h * (1.0 + eps_ref[0, 0]) + agg_ref[0, :N, :] + agg_ref[1, :N, :]
    y = jnp.dot(z.astype(jnp.bfloat16), wm1[...],
                preferred_element_type=jnp.float32) + bm1[...]
    mean = jnp.mean(y, axis=0, keepdims=True)
    var = jnp.mean(jnp.square(y - mean), axis=0, keepdims=True)
    y = (y - mean) / jnp.sqrt(var + 1e-5) * gm[...] + bt[...]
    y = jnp.maximum(y, 0.0)
    cc = jnp.dot(y.astype(jnp.bfloat16), wm2[...],
                 preferred_element_type=jnp.float32) + bm2[...]
    hn = jnp.maximum(cc, 0.0) if first else h + jnp.maximum(cc, 0.0)
    if last:
        o = jnp.dot(hn, wa[...], preferred_element_type=jnp.float32) + ba[...]
        sp = jnp.maximum(o, 0.0) + jnp.log1p(jnp.exp(-jnp.abs(o)))
        col = lax.broadcasted_iota(jnp.int32, o.shape, 1)
        out_refs[0][...] = jnp.where(col == 0, o, sp)
    else:
        out_refs[0][...] = hn
        out_refs[1][...] = hn + benext[...]


def _mlp(first, last, h, agg, p, Wa, ba, be_next):
    body = functools.partial(_mlp_body, first, last)
    if last:
        shapes = jax.ShapeDtypeStruct((N, 2), jnp.float32)
    else:
        shapes = [jax.ShapeDtypeStruct((N, H), jnp.float32),
                  jax.ShapeDtypeStruct((N, H), jnp.float32)]
    return pl.pallas_call(
        body,
        out_shape=shapes,
    )(h, agg, p['eps'].reshape(1, 1),
      p['Wm1'].astype(jnp.bfloat16), p['bm1'].reshape(1, H),
      p['gamma'].reshape(1, H), p['beta'].reshape(1, H),
      p['Wm2'].astype(jnp.bfloat16), p['bm2'].reshape(1, H),
      Wa, ba.reshape(1, 2), be_next.reshape(1, H))


def kernel(ensemble, x, edge_index, edge_attr, deepset, Wd, bd, convs, Wa, ba):
    nf, nfb = _pre(ensemble, x, deepset, Wd, bd, convs[0]['be'])

    pad_i = jnp.arange(E_PAD - E, dtype=jnp.int32)
    src = jnp.concatenate(
        [edge_index[0], pad_i % N]
    ).reshape(TOTCH, C)
    dst3 = jnp.concatenate(
        [edge_index[1], N + (pad_i % 128)]
    ).reshape(TOTCH, C)
    ea = jnp.concatenate(
        [edge_attr[:, 0], jnp.zeros((E_PAD - E,), jnp.float32)]
    ).reshape(TOTCH, C)
    zeros = jnp.zeros((N_PAD, H), jnp.float32)

    h, hb = nf, nfb
    for i, p in enumerate(convs):
        last = i == len(convs) - 1
        out = _mp_call()(hb, src, dst3, ea, p['We'], zeros)
        res = _mlp(i == 0, last, h, out, p, Wa, ba,
                   p['be'] if last else convs[i + 1]['be'])
        if last:
            return res
        h, hb = res


# speedup vs baseline: 1.0640x; 1.0640x over previous
import functools

import jax
import jax.numpy as jnp
from jax import lax
from jax.experimental import pallas as pl
from jax.experimental.pallas import tpu as pltpu
from jax.experimental.pallas import tpu_sc as plsc

N = 10000
E = 320000
D_IN = 128
H = 64
ENS = 10

NB = 1000
GRID_PRE = N // NB

C = 128
NW = 32
TOTCH = 2560
E_PAD = TOTCH * C
CPW0 = 80
CPW1 = 80
CPWMAX = max(CPW0, CPW1)
NBUF = 4
LOOK = 2
SUB_ROWS = 648
N_PAD = SUB_ROWS * 16


def _pre_body(ens_ref, x_ref, w1, b1, w2, b2, w3, b3, w4, b4, wdx, wde, bd,
              out_ref):
    sacc = jnp.zeros((NB, H), jnp.float32)
    for e in range(ENS):
        m = ens_ref[e].astype(jnp.bfloat16)
        sacc = sacc + jnp.maximum(
            jnp.dot(m, w1[...], preferred_element_type=jnp.float32) + b1[...],
            0.0)
    agg = (jnp.dot(sacc.astype(jnp.bfloat16), w2[...],
                   preferred_element_type=jnp.float32)
           + float(ENS) * b2[...])
    emb = jnp.maximum(jnp.dot(agg, w3[...], preferred_element_type=jnp.float32)
                      + b3[...], 0.0)
    emb = jnp.dot(emb, w4[...], preferred_element_type=jnp.float32) + b4[...]
    nf = (jnp.dot(x_ref[...], wdx[...], preferred_element_type=jnp.float32)
          + jnp.dot(emb, wde[...], preferred_element_type=jnp.float32)
          + bd[...])
    out_ref[...] = nf


def _pre(ensemble, x, ds, Wd, bd):
    full = lambda shape: pl.BlockSpec(shape, lambda i: (0,) * len(shape))
    return pl.pallas_call(
        _pre_body,
        grid=(GRID_PRE,),
        in_specs=[
            pl.BlockSpec((ENS, NB, D_IN), lambda i: (0, i, 0)),
            pl.BlockSpec((NB, D_IN), lambda i: (i, 0)),
            full((D_IN, H)), full((1, H)),
            full((H, H)), full((1, H)),
            full((H, H)), full((1, H)),
            full((H, H)), full((1, H)),
            full((D_IN, H)), full((H, H)), full((1, H)),
        ],
        out_specs=pl.BlockSpec((NB, H), lambda i: (i, 0)),
        out_shape=jax.ShapeDtypeStruct((N, H), jnp.float32),
    )(ensemble.transpose(1, 0, 2), x,
      ds['W1'].astype(jnp.bfloat16), ds['b1'].reshape(1, H),
      ds['W2'].astype(jnp.bfloat16), ds['b2'].reshape(1, H),
      ds['W3'], ds['b3'].reshape(1, H),
      ds['W4'], ds['b4'].reshape(1, H),
      Wd[:D_IN], Wd[D_IN:], bd.reshape(1, H))


def _mp_body(h_hbm, src_hbm, dst_hbm, ea_hbm, wb_hbm, z_hbm, out_hbm,
             srcv, dstv, eav, rowsv, wbv, acc, gsem, ssem):
    c = lax.axis_index("c")
    s = lax.axis_index("s")

    with jax.named_scope("zero_acc"):
        pltpu.sync_copy(z_hbm.at[pl.ds(s * SUB_ROWS, SUB_ROWS)],
                        acc.at[pl.ds(s * SUB_ROWS, SUB_ROWS)])
        pltpu.sync_copy(wb_hbm, wbv)

    def fire_gather(j, b):
        pltpu.async_copy(h_hbm.at[srcv.at[j]], rowsv.at[b], gsem.at[b])

    def wait_gather(j, b):
        pltpu.make_async_copy(h_hbm.at[srcv.at[j]], rowsv.at[b],
                              gsem.at[b]).wait()

    def fire_scatter(j, b):
        pltpu.async_copy(rowsv.at[b], acc.at[dstv.at[j]], ssem.at[b],
                         add=True)

    def wait_scatter(j, b):
        pltpu.make_async_copy(rowsv.at[b], acc.at[dstv.at[j]],
                              ssem.at[b]).wait()

    we = [wbv[0, pl.ds(g * 16, 16)] for g in range(4)]
    be = [wbv[1, pl.ds(g * 16, 16)] for g in range(4)]

    def run(start, cpw):
        with jax.named_scope("stage_idx"):
            pltpu.sync_copy(src_hbm.at[pl.ds(start, cpw)],
                            srcv.at[pl.ds(0, cpw)])
            pltpu.sync_copy(dst_hbm.at[pl.ds(start, cpw)],
                            dstv.at[pl.ds(0, cpw)])
            pltpu.sync_copy(ea_hbm.at[pl.ds(start, cpw)],
                            eav.at[pl.ds(0, cpw)])

        for b in range(LOOK):
            fire_gather(b, b)

        with jax.named_scope("pre_barrier"):
            plsc.subcore_barrier()

        @pl.loop(0, cpw // NBUF)
        def _outer(jo):
            for b in range(NBUF):
                j = jo * NBUF + b
                bn = (b + LOOK) % NBUF

                @pl.when(j + LOOK < cpw)
                def _fire():
                    @pl.when(j + LOOK >= NBUF)
                    def _drain():
                        wait_scatter(j + LOOK - NBUF, bn)
                    fire_gather(j + LOOK, bn)

                wait_gather(j, b)

                @pl.loop(0, C // 16)
                def _blk(jj):
                    a_vec = eav[j, pl.ds(jj * 16, 16)]
                    for ii in range(16):
                        i = jj * 16 + ii
                        a = a_vec[ii]
                        for g in range(4):
                            sl = pl.ds(g * 16, 16)
                            rowsv[b, i, sl] = jnp.maximum(
                                rowsv[b, i, sl] + (we[g] * a + be[g]), 0.0)

                fire_scatter(j, b)

        with jax.named_scope("drain"):
            for b in range(NBUF):
                wait_scatter(cpw - NBUF + b, b)

    @pl.when(c == 0)
    def _core0():
        run(s * CPW0, CPW0)

    @pl.when(c == 1)
    def _core1():
        run(16 * CPW0 + s * CPW1, CPW1)

    with jax.named_scope("post_barrier"):
        plsc.subcore_barrier()
    with jax.named_scope("writeout"):
        pltpu.sync_copy(acc.at[pl.ds(s * SUB_ROWS, SUB_ROWS)],
                        out_hbm.at[c, pl.ds(s * SUB_ROWS, SUB_ROWS)])


@functools.lru_cache(maxsize=None)
def _mp_call():
    return pl.kernel(
        _mp_body,
        mesh=plsc.VectorSubcoreMesh(core_axis_name="c", subcore_axis_name="s"),
        out_type=jax.ShapeDtypeStruct((2, N_PAD, H), jnp.float32),
        scratch_types=[
            pltpu.VMEM((CPWMAX, C), jnp.int32),
            pltpu.VMEM((CPWMAX, C), jnp.int32),
            pltpu.VMEM((CPWMAX, C), jnp.float32),
            pltpu.VMEM((NBUF, C, H), jnp.float32),
            pltpu.VMEM((2, H), jnp.float32),
            pltpu.VMEM_SHARED((N_PAD, H), jnp.float32),
            pltpu.SemaphoreType.DMA((NBUF,)),
            pltpu.SemaphoreType.DMA((NBUF,)),
        ],
        compiler_params=pltpu.CompilerParams(use_tc_tiling_on_sc=False),
    )


def _mlp_body(first, last, h_ref, agg_ref, eps_ref, wm1, bm1, gm, bt,
              wm2, bm2, wa, ba, *out_refs):
    h = h_ref[...]
    z = h * (1.0 + eps_ref[0, 0]) + agg_ref[0, :N, :] + agg_ref[1, :N, :]
    y = jnp.dot(z.astype(jnp.bfloat16), wm1[...],
                preferred_element_type=jnp.float32) + bm1[...]
    mean = jnp.mean(y, axis=0, keepdims=True)
    var = jnp.mean(jnp.square(y - mean), axis=0, keepdims=True)
    y = (y - mean) / jnp.sqrt(var + 1e-5) * gm[...] + bt[...]
    y = jnp.maximum(y, 0.0)
    cc = jnp.dot(y.astype(jnp.bfloat16), wm2[...],
                 preferred_element_type=jnp.float32) + bm2[...]
    hn = jnp.maximum(cc, 0.0) if first else h + jnp.maximum(cc, 0.0)
    if last:
        o = jnp.dot(hn, wa[...], preferred_element_type=jnp.float32) + ba[...]
        sp = jnp.maximum(o, 0.0) + jnp.log1p(jnp.exp(-jnp.abs(o)))
        col = lax.broadcasted_iota(jnp.int32, o.shape, 1)
        out_refs[0][...] = jnp.where(col == 0, o, sp)
    else:
        out_refs[0][...] = hn


def _mlp(first, last, h, agg, p, Wa, ba):
    body = functools.partial(_mlp_body, first, last)
    odim = 2 if last else H
    return pl.pallas_call(
        body,
        out_shape=jax.ShapeDtypeStruct((N, odim), jnp.float32),
    )(h, agg, p['eps'].reshape(1, 1),
      p['Wm1'].astype(jnp.bfloat16), p['bm1'].reshape(1, H),
      p['gamma'].reshape(1, H), p['beta'].reshape(1, H),
      p['Wm2'].astype(jnp.bfloat16), p['bm2'].reshape(1, H),
      Wa, ba.reshape(1, 2))


def kernel(ensemble, x, edge_index, edge_attr, deepset, Wd, bd, convs, Wa, ba):
    nf = _pre(ensemble, x, deepset, Wd, bd)

    pad_i = jnp.arange(E_PAD - E, dtype=jnp.int32)
    src = jnp.concatenate(
        [edge_index[0], pad_i % N]
    ).reshape(TOTCH, C)
    dst3 = jnp.concatenate(
        [edge_index[1], N + (pad_i % 128)]
    ).reshape(TOTCH, C)
    ea = jnp.concatenate(
        [edge_attr[:, 0], jnp.zeros((E_PAD - E,), jnp.float32)]
    ).reshape(TOTCH, C)
    zeros = jnp.zeros((N_PAD, H), jnp.float32)

    h = nf
    for i, p in enumerate(convs):
        wb = jnp.stack([p['We'][0], p['be']])
        out = _mp_call()(h, src, dst3, ea, wb, zeros)
        h = _mlp(i == 0, i == len(convs) - 1, h, out, p, Wa, ba)
    return h


# speedup vs baseline: 1.0881x; 1.0226x over previous
import functools

import jax
import jax.numpy as jnp
from jax import lax
from jax.experimental import pallas as pl
from jax.experimental.pallas import tpu as pltpu
from jax.experimental.pallas import tpu_sc as plsc

N = 10000
E = 320000
D_IN = 128
H = 64
ENS = 10

NB = 1000
GRID_PRE = N // NB

C = 128
NW = 32
TOTCH = 2560
E_PAD = TOTCH * C
CPW0 = 80
CPW1 = 80
CPWMAX = max(CPW0, CPW1)
NBUF = 4
LOOK = 2
SUB_ROWS = 648
N_PAD = SUB_ROWS * 16


def _pre_body(ens_ref, x_ref, w1, b1, w2, b2, w3, b3, w4, b4, wdx, wde, bd,
              out_ref):
    sacc = jnp.zeros((NB, H), jnp.float32)
    for e in range(ENS):
        m = ens_ref[e].astype(jnp.bfloat16)
        sacc = sacc + jnp.maximum(
            jnp.dot(m, w1[...], preferred_element_type=jnp.float32) + b1[...],
            0.0)
    agg = (jnp.dot(sacc.astype(jnp.bfloat16), w2[...],
                   preferred_element_type=jnp.float32)
           + float(ENS) * b2[...])
    emb = jnp.maximum(jnp.dot(agg, w3[...], preferred_element_type=jnp.float32)
                      + b3[...], 0.0)
    emb = jnp.dot(emb, w4[...], preferred_element_type=jnp.float32) + b4[...]
    nf = (jnp.dot(x_ref[...], wdx[...], preferred_element_type=jnp.float32)
          + jnp.dot(emb, wde[...], preferred_element_type=jnp.float32)
          + bd[...])
    out_ref[...] = nf


def _pre(ensemble, x, ds, Wd, bd):
    full = lambda shape: pl.BlockSpec(shape, lambda i: (0,) * len(shape))
    return pl.pallas_call(
        _pre_body,
        grid=(GRID_PRE,),
        in_specs=[
            pl.BlockSpec((ENS, NB, D_IN), lambda i: (0, i, 0)),
            pl.BlockSpec((NB, D_IN), lambda i: (i, 0)),
            full((D_IN, H)), full((1, H)),
            full((H, H)), full((1, H)),
            full((H, H)), full((1, H)),
            full((H, H)), full((1, H)),
            full((D_IN, H)), full((H, H)), full((1, H)),
        ],
        out_specs=pl.BlockSpec((NB, H), lambda i: (i, 0)),
        out_shape=jax.ShapeDtypeStruct((N, H), jnp.float32),
    )(ensemble.transpose(1, 0, 2), x,
      ds['W1'].astype(jnp.bfloat16), ds['b1'].reshape(1, H),
      ds['W2'].astype(jnp.bfloat16), ds['b2'].reshape(1, H),
      ds['W3'], ds['b3'].reshape(1, H),
      ds['W4'], ds['b4'].reshape(1, H),
      Wd[:D_IN], Wd[D_IN:], bd.reshape(1, H))


def _mp_body(h_hbm, src_hbm, dst_hbm, ea_hbm, wb_hbm, z_hbm, out_hbm,
             srcv, dstv, eav, rowsv, wbv, acc, gsem, ssem):
    c = lax.axis_index("c")
    s = lax.axis_index("s")

    with jax.named_scope("zero_acc"):
        pltpu.sync_copy(z_hbm.at[pl.ds(s * SUB_ROWS, SUB_ROWS)],
                        acc.at[pl.ds(s * SUB_ROWS, SUB_ROWS)])
        pltpu.sync_copy(wb_hbm, wbv)

    def fire_gather(j, b):
        pltpu.async_copy(h_hbm.at[srcv.at[j]], rowsv.at[b], gsem.at[b])

    def wait_gather(j, b):
        pltpu.make_async_copy(h_hbm.at[srcv.at[j]], rowsv.at[b],
                              gsem.at[b]).wait()

    def fire_scatter(j, b):
        pltpu.async_copy(rowsv.at[b], acc.at[dstv.at[j]], ssem.at[b],
                         add=True)

    def wait_scatter(j, b):
        pltpu.make_async_copy(rowsv.at[b], acc.at[dstv.at[j]],
                              ssem.at[b]).wait()

    we = [wbv[0, pl.ds(g * 16, 16)] for g in range(4)]
    be = [wbv[1, pl.ds(g * 16, 16)] for g in range(4)]

    def run(start, cpw):
        with jax.named_scope("stage_idx"):
            pltpu.sync_copy(src_hbm.at[pl.ds(start, cpw)],
                            srcv.at[pl.ds(0, cpw)])
            pltpu.sync_copy(dst_hbm.at[pl.ds(start, cpw)],
                            dstv.at[pl.ds(0, cpw)])
            pltpu.sync_copy(ea_hbm.at[pl.ds(start, cpw)],
                            eav.at[pl.ds(0, cpw)])

        for b in range(LOOK):
            fire_gather(b, b)

        with jax.named_scope("pre_barrier"):
            plsc.subcore_barrier()

        @pl.loop(0, cpw // NBUF)
        def _outer(jo):
            for b in range(NBUF):
                j = jo * NBUF + b
                bn = (b + LOOK) % NBUF

                @pl.when(j + LOOK < cpw)
                def _fire():
                    @pl.when(j + LOOK >= NBUF)
                    def _drain():
                        wait_scatter(j + LOOK - NBUF, bn)
                    fire_gather(j + LOOK, bn)

                wait_gather(j, b)

                @pl.loop(0, C // 16)
                def _blk(jj):
                    a_vec = eav[j, pl.ds(jj * 16, 16)]
                    for ii in range(16):
                        i = jj * 16 + ii
                        a = a_vec[ii]
                        for g in range(4):
                            sl = pl.ds(g * 16, 16)
                            rowsv[b, i, sl] = jnp.maximum(
                                rowsv[b, i, sl] + (we[g] * a + be[g]), 0.0)

                fire_scatter(j, b)

        with jax.named_scope("drain"):
            for b in range(NBUF):
                wait_scatter(cpw - NBUF + b, b)

    @pl.when(c == 0)
    def _core0():
        run(s * CPW0, CPW0)

    @pl.when(c == 1)
    def _core1():
        run(16 * CPW0 + s * CPW1, CPW1)

    with jax.named_scope("post_barrier"):
        plsc.subcore_barrier()
    with jax.named_scope("writeout"):
        pltpu.sync_copy(acc.at[pl.ds(s * SUB_ROWS, SUB_ROWS)],
                        out_hbm.at[c, pl.ds(s * SUB_ROWS, SUB_ROWS)])


@functools.lru_cache(maxsize=None)
def _mp_call():
    return pl.kernel(
        _mp_body,
        mesh=plsc.VectorSubcoreMesh(core_axis_name="c", subcore_axis_name="s"),
        out_type=jax.ShapeDtypeStruct((2, N_PAD, H), jnp.float32),
        scratch_types=[
            pltpu.VMEM((CPWMAX, C), jnp.int32),
            pltpu.VMEM((CPWMAX, C), jnp.int32),
            pltpu.VMEM((CPWMAX, C), jnp.float32),
            pltpu.VMEM((NBUF, C, H), jnp.float32),
            pltpu.VMEM((2, H), jnp.float32),
            pltpu.VMEM_SHARED((N_PAD, H), jnp.float32),
            pltpu.SemaphoreType.DMA((NBUF,)),
            pltpu.SemaphoreType.DMA((NBUF,)),
        ],
        compiler_params=pltpu.CompilerParams(use_tc_tiling_on_sc=False),
    )


def _mlp_body(first, last, h_ref, agg_ref, eps_ref, wm1, bm1, gm, bt,
              wm2, bm2, wa, ba, *out_refs):
    h = h_ref[...]
    z = h * (1.0 + eps_ref[0, 0]) + agg_ref[0, :N, :] + agg_ref[1, :N, :]
    y = jnp.dot(z, wm1[...], preferred_element_type=jnp.float32) + bm1[...]
    mean = jnp.mean(y, axis=0, keepdims=True)
    var = jnp.mean(jnp.square(y - mean), axis=0, keepdims=True)
    y = (y - mean) / jnp.sqrt(var + 1e-5) * gm[...] + bt[...]
    y = jnp.maximum(y, 0.0)
    cc = jnp.dot(y, wm2[...], preferred_element_type=jnp.float32) + bm2[...]
    hn = jnp.maximum(cc, 0.0) if first else h + jnp.maximum(cc, 0.0)
    if last:
        o = jnp.dot(hn, wa[...], preferred_element_type=jnp.float32) + ba[...]
        sp = jnp.maximum(o, 0.0) + jnp.log1p(jnp.exp(-jnp.abs(o)))
        col = lax.broadcasted_iota(jnp.int32, o.shape, 1)
        out_refs[0][...] = jnp.where(col == 0, o, sp)
    else:
        out_refs[0][...] = hn


def _mlp(first, last, h, agg, p, Wa, ba):
    body = functools.partial(_mlp_body, first, last)
    odim = 2 if last else H
    return pl.pallas_call(
        body,
        out_shape=jax.ShapeDtypeStruct((N, odim), jnp.float32),
    )(h, agg, p['eps'].reshape(1, 1),
      p['Wm1'], p['bm1'].reshape(1, H),
      p['gamma'].reshape(1, H), p['beta'].reshape(1, H),
      p['Wm2'], p['bm2'].reshape(1, H),
      Wa, ba.reshape(1, 2))


def kernel(ensemble, x, edge_index, edge_attr, deepset, Wd, bd, convs, Wa, ba):
    nf = _pre(ensemble, x, deepset, Wd, bd)

    pad_i = jnp.arange(E_PAD - E, dtype=jnp.int32)
    src = jnp.concatenate(
        [edge_index[0], pad_i % N]
    ).reshape(TOTCH, C)
    dst3 = jnp.concatenate(
        [edge_index[1], N + (pad_i % 128)]
    ).reshape(TOTCH, C)
    ea = jnp.concatenate(
        [edge_attr[:, 0], jnp.zeros((E_PAD - E,), jnp.float32)]
    ).reshape(TOTCH, C)
    zeros = jnp.zeros((N_PAD, H), jnp.float32)

    h = nf
    for i, p in enumerate(convs):
        wb = jnp.stack([p['We'][0], p['be']])
        out = _mp_call()(h, src, dst3, ea, wb, zeros)
        h = _mlp(i == 0, i == len(convs) - 1, h, out, p, Wa, ba)
    return h


# speedup vs baseline: 1.0946x; 1.0059x over previous
import functools

import jax
import jax.numpy as jnp
from jax import lax
from jax.experimental import pallas as pl
from jax.experimental.pallas import tpu as pltpu
from jax.experimental.pallas import tpu_sc as plsc

N = 10000
E = 320000
D_IN = 128
H = 64
ENS = 10

NB = 2000
GRID_PRE = N // NB

C = 128
NW = 32
TOTCH = 2560
E_PAD = TOTCH * C
CPW0 = 80
CPW1 = 80
CPWMAX = max(CPW0, CPW1)
NBUF = 5
LOOK = 2
SUB_ROWS = 648
N_PAD = SUB_ROWS * 16


def _pre_body(ens_ref, x_ref, w1, b1, w2, b2, w3, b3, w4, b4, wdx, wde, bd,
              out_ref):
    sacc = jnp.zeros((NB, H), jnp.float32)
    for e in range(ENS):
        m = ens_ref[e].astype(jnp.bfloat16)
        sacc = sacc + jnp.maximum(
            jnp.dot(m, w1[...], preferred_element_type=jnp.float32) + b1[...],
            0.0)
    agg = (jnp.dot(sacc.astype(jnp.bfloat16), w2[...],
                   preferred_element_type=jnp.float32)
           + float(ENS) * b2[...])
    emb = jnp.maximum(jnp.dot(agg, w3[...], preferred_element_type=jnp.float32)
                      + b3[...], 0.0)
    emb = jnp.dot(emb, w4[...], preferred_element_type=jnp.float32) + b4[...]
    nf = (jnp.dot(x_ref[...], wdx[...], preferred_element_type=jnp.float32)
          + jnp.dot(emb, wde[...], preferred_element_type=jnp.float32)
          + bd[...])
    out_ref[...] = nf


def _pre(ensemble, x, ds, Wd, bd):
    full = lambda shape: pl.BlockSpec(shape, lambda i: (0,) * len(shape))
    return pl.pallas_call(
        _pre_body,
        grid=(GRID_PRE,),
        in_specs=[
            pl.BlockSpec((ENS, NB, D_IN), lambda i: (0, i, 0)),
            pl.BlockSpec((NB, D_IN), lambda i: (i, 0)),
            full((D_IN, H)), full((1, H)),
            full((H, H)), full((1, H)),
            full((H, H)), full((1, H)),
            full((H, H)), full((1, H)),
            full((D_IN, H)), full((H, H)), full((1, H)),
        ],
        out_specs=pl.BlockSpec((NB, H), lambda i: (i, 0)),
        out_shape=jax.ShapeDtypeStruct((N, H), jnp.float32),
    )(ensemble.transpose(1, 0, 2), x,
      ds['W1'].astype(jnp.bfloat16), ds['b1'].reshape(1, H),
      ds['W2'].astype(jnp.bfloat16), ds['b2'].reshape(1, H),
      ds['W3'], ds['b3'].reshape(1, H),
      ds['W4'], ds['b4'].reshape(1, H),
      Wd[:D_IN], Wd[D_IN:], bd.reshape(1, H))


def _mp_body(h_hbm, src_hbm, dst_hbm, ea_hbm, wb_hbm, z_hbm, out_hbm,
             srcv, dstv, eav, rowsv, wbv, acc, gsem, ssem):
    c = lax.axis_index("c")
    s = lax.axis_index("s")

    with jax.named_scope("zero_acc"):
        pltpu.sync_copy(z_hbm.at[pl.ds(s * SUB_ROWS, SUB_ROWS)],
                        acc.at[pl.ds(s * SUB_ROWS, SUB_ROWS)])
        pltpu.sync_copy(wb_hbm, wbv)

    def fire_gather(j, b):
        pltpu.async_copy(h_hbm.at[srcv.at[j]], rowsv.at[b], gsem.at[b])

    def wait_gather(j, b):
        pltpu.make_async_copy(h_hbm.at[srcv.at[j]], rowsv.at[b],
                              gsem.at[b]).wait()

    def fire_scatter(j, b):
        pltpu.async_copy(rowsv.at[b], acc.at[dstv.at[j]], ssem.at[b],
                         add=True)

    def wait_scatter(j, b):
        pltpu.make_async_copy(rowsv.at[b], acc.at[dstv.at[j]],
                              ssem.at[b]).wait()

    we = [wbv[0, pl.ds(g * 16, 16)] for g in range(4)]
    be = [wbv[1, pl.ds(g * 16, 16)] for g in range(4)]

    def run(start, cpw):
        with jax.named_scope("stage_idx"):
            pltpu.sync_copy(src_hbm.at[pl.ds(start, cpw)],
                            srcv.at[pl.ds(0, cpw)])
            pltpu.sync_copy(dst_hbm.at[pl.ds(start, cpw)],
                            dstv.at[pl.ds(0, cpw)])
            pltpu.sync_copy(ea_hbm.at[pl.ds(start, cpw)],
                            eav.at[pl.ds(0, cpw)])

        for b in range(LOOK):
            fire_gather(b, b)

        with jax.named_scope("pre_barrier"):
            plsc.subcore_barrier()

        @pl.loop(0, cpw // NBUF)
        def _outer(jo):
            for b in range(NBUF):
                j = jo * NBUF + b
                bn = (b + LOOK) % NBUF

                @pl.when(j + LOOK < cpw)
                def _fire():
                    @pl.when(j + LOOK >= NBUF)
                    def _drain():
                        wait_scatter(j + LOOK - NBUF, bn)
                    fire_gather(j + LOOK, bn)

                wait_gather(j, b)

                @pl.loop(0, C // 16)
                def _blk(jj):
                    a_vec = eav[j, pl.ds(jj * 16, 16)]
                    for ii in range(16):
                        i = jj * 16 + ii
                        a = a_vec[ii]
                        for g in range(4):
                            sl = pl.ds(g * 16, 16)
                            rowsv[b, i, sl] = jnp.maximum(
                                rowsv[b, i, sl] + (we[g] * a + be[g]), 0.0)

                fire_scatter(j, b)

        with jax.named_scope("drain"):
            for b in range(NBUF):
                wait_scatter(cpw - NBUF + b, b)

    @pl.when(c == 0)
    def _core0():
        run(s * CPW0, CPW0)

    @pl.when(c == 1)
    def _core1():
        run(16 * CPW0 + s * CPW1, CPW1)

    with jax.named_scope("post_barrier"):
        plsc.subcore_barrier()
    with jax.named_scope("writeout"):
        pltpu.sync_copy(acc.at[pl.ds(s * SUB_ROWS, SUB_ROWS)],
                        out_hbm.at[c, pl.ds(s * SUB_ROWS, SUB_ROWS)])


@functools.lru_cache(maxsize=None)
def _mp_call():
    return pl.kernel(
        _mp_body,
        mesh=plsc.VectorSubcoreMesh(core_axis_name="c", subcore_axis_name="s"),
        out_type=jax.ShapeDtypeStruct((2, N_PAD, H), jnp.float32),
        scratch_types=[
            pltpu.VMEM((CPWMAX, C), jnp.int32),
            pltpu.VMEM((CPWMAX, C), jnp.int32),
            pltpu.VMEM((CPWMAX, C), jnp.float32),
            pltpu.VMEM((NBUF, C, H), jnp.float32),
            pltpu.VMEM((2, H), jnp.float32),
            pltpu.VMEM_SHARED((N_PAD, H), jnp.float32),
            pltpu.SemaphoreType.DMA((NBUF,)),
            pltpu.SemaphoreType.DMA((NBUF,)),
        ],
        compiler_params=pltpu.CompilerParams(use_tc_tiling_on_sc=False),
    )


def _mlp_body(first, last, h_ref, agg_ref, eps_ref, wm1, bm1, gm, bt,
              wm2, bm2, wa, ba, *out_refs):
    h = h_ref[...]
    z = h * (1.0 + eps_ref[0, 0]) + agg_ref[0, :N, :] + agg_ref[1, :N, :]
    y = jnp.dot(z, wm1[...], preferred_element_type=jnp.float32) + bm1[...]
    mean = jnp.mean(y, axis=0, keepdims=True)
    var = jnp.mean(jnp.square(y - mean), axis=0, keepdims=True)
    y = (y - mean) / jnp.sqrt(var + 1e-5) * gm[...] + bt[...]
    y = jnp.maximum(y, 0.0)
    cc = jnp.dot(y, wm2[...], preferred_element_type=jnp.float32) + bm2[...]
    hn = jnp.maximum(cc, 0.0) if first else h + jnp.maximum(cc, 0.0)
    if last:
        o = jnp.dot(hn, wa[...], preferred_element_type=jnp.float32) + ba[...]
        sp = jnp.maximum(o, 0.0) + jnp.log1p(jnp.exp(-jnp.abs(o)))
        col = lax.broadcasted_iota(jnp.int32, o.shape, 1)
        out_refs[0][...] = jnp.where(col == 0, o, sp)
    else:
        out_refs[0][...] = hn


def _mlp(first, last, h, agg, p, Wa, ba):
    body = functools.partial(_mlp_body, first, last)
    odim = 2 if last else H
    return pl.pallas_call(
        body,
        out_shape=jax.ShapeDtypeStruct((N, odim), jnp.float32),
    )(h, agg, p['eps'].reshape(1, 1),
      p['Wm1'], p['bm1'].reshape(1, H),
      p['gamma'].reshape(1, H), p['beta'].reshape(1, H),
      p['Wm2'], p['bm2'].reshape(1, H),
      Wa, ba.reshape(1, 2))


def kernel(ensemble, x, edge_index, edge_attr, deepset, Wd, bd, convs, Wa, ba):
    nf = _pre(ensemble, x, deepset, Wd, bd)

    pad_i = jnp.arange(E_PAD - E, dtype=jnp.int32)
    src = jnp.concatenate(
        [edge_index[0], pad_i % N]
    ).reshape(TOTCH, C)
    dst3 = jnp.concatenate(
        [edge_index[1], N + (pad_i % 128)]
    ).reshape(TOTCH, C)
    ea = jnp.concatenate(
        [edge_attr[:, 0], jnp.zeros((E_PAD - E,), jnp.float32)]
    ).reshape(TOTCH, C)
    zeros = jnp.zeros((N_PAD, H), jnp.float32)

    h = nf
    for i, p in enumerate(convs):
        wb = jnp.stack([p['We'][0], p['be']])
        out = _mp_call()(h, src, dst3, ea, wb, zeros)
        h = _mlp(i == 0, i == len(convs) - 1, h, out, p, Wa, ba)
    return h
